# baseline (speedup 1.0000x reference)
import functools

import jax
import jax.numpy as jnp
from jax import lax
from jax.experimental import pallas as pl
from jax.experimental.pallas import tpu as pltpu
from jax.experimental.pallas import tpu_sc as plsc

N = 10000
E = 320000
D_IN = 128
D_H = 128
D_OUT = 40
D_OUT_PAD = 128

NC = 2
NS = 16
CHUNK = 128
CHUNKS_PER_SUB = 80
NPHASE = 2
HALF = CHUNKS_PER_SUB // NPHASE
EPAD = NC * NS * CHUNKS_PER_SUB * CHUNK
NPAD = 10240
ROWS_PER_SUB = NPAD // NS
DEG_W = 16

_VMESH = plsc.VectorSubcoreMesh(core_axis_name="c", subcore_axis_name="s")


def _row_blocks():
    off = 0
    blocks = []
    while off < ROWS_PER_SUB:
        r = min(CHUNK, ROWS_PER_SUB - off)
        blocks.append((off, r))
        off += r
    return blocks


def _fill_rows(ref, value):
    rows, cols = ref.shape
    v = jnp.full((16,), value, ref.dtype)

    @pl.loop(0, rows)
    def _(i):
        for j in range(cols // 16):
            ref[i, pl.ds(j * 16, 16)] = v


@functools.lru_cache(maxsize=None)
def _sc_scatter_rows(width):
    @functools.partial(
        pl.kernel,
        out_type=jax.ShapeDtypeStruct((NC, NPAD, width), jnp.float32),
        mesh=_VMESH,
        scratch_types=[
            pltpu.VMEM((HALF, CHUNK), jnp.int32),
            pltpu.VMEM((HALF, CHUNK), jnp.int32),
            pltpu.VMEM((CHUNK, width), jnp.float32),
            pltpu.VMEM((CHUNK, width), jnp.float32),
            pltpu.VMEM_SHARED((NPAD, width), jnp.float32),
            pltpu.SemaphoreType.DMA,
            pltpu.SemaphoreType.DMA,
        ],
    )
    def k(h_hbm, src_hbm, dst_hbm, z_hbm, out_hbm, src_v, dst_v, buf0, buf1,
          acc, sem0, sem1):
        c = lax.axis_index("c")
        s = lax.axis_index("s")
        base = s * ROWS_PER_SUB

        def gather(j, buf, sem):
            pltpu.async_copy(h_hbm.at[src_v.at[j]], buf, sem)

        def gather_wait(buf, sem):
            pltpu.make_async_copy(h_hbm.at[src_v.at[0]], buf, sem).wait()

        pltpu.sync_copy(z_hbm, acc.at[pl.ds(base, ROWS_PER_SUB)])
        plsc.subcore_barrier()

        for p in range(NPHASE):
            pltpu.sync_copy(src_hbm.at[c, s, p], src_v)
            pltpu.sync_copy(dst_hbm.at[c, s, p], dst_v)
            gather(0, buf0, sem0)
            gather(1, buf1, sem1)

            @pl.loop(0, HALF, step=2)
            def _(j):
                gather_wait(buf0, sem0)
                pltpu.sync_copy(buf0, acc.at[dst_v.at[j]], add=True)

                @pl.when(j + 2 < HALF)
                def _():
                    gather(j + 2, buf0, sem0)

                gather_wait(buf1, sem1)
                pltpu.sync_copy(buf1, acc.at[dst_v.at[j + 1]], add=True)

                @pl.when(j + 3 < HALF)
                def _():
                    gather(j + 3, buf1, sem1)

        plsc.subcore_barrier()
        pltpu.sync_copy(acc.at[pl.ds(base, ROWS_PER_SUB)],
                        out_hbm.at[c, pl.ds(base, ROWS_PER_SUB)])

    return k


@functools.lru_cache(maxsize=None)
def _sc_degree():
    @functools.partial(
        pl.kernel,
        out_type=jax.ShapeDtypeStruct((NC, NPAD, DEG_W), jnp.float32),
        mesh=_VMESH,
        scratch_types=[
            pltpu.VMEM((CHUNKS_PER_SUB, CHUNK), jnp.int32),
            pltpu.VMEM((CHUNK, DEG_W), jnp.float32),
            pltpu.VMEM_SHARED((NPAD, DEG_W), jnp.float32),
        ],
    )
    def k(dst_hbm, zd_hbm, ones_hbm, out_hbm, dst_v, ones_v, acc):
        c = lax.axis_index("c")
        s = lax.axis_index("s")
        base = s * ROWS_PER_SUB
        pltpu.sync_copy(zd_hbm, acc.at[pl.ds(base, ROWS_PER_SUB)])
        pltpu.sync_copy(ones_hbm, ones_v)
        pltpu.sync_copy(dst_hbm.at[c, s], dst_v)
        plsc.subcore_barrier()

        @pl.loop(0, CHUNKS_PER_SUB)
        def _(j):
            pltpu.sync_copy(ones_v, acc.at[dst_v.at[j]], add=True)

        plsc.subcore_barrier()
        pltpu.sync_copy(acc.at[pl.ds(base, ROWS_PER_SUB)],
                        out_hbm.at[c, pl.ds(base, ROWS_PER_SUB)])

    return k


def _dot(a, b):
    return lax.dot_general(a, b, (((1,), (0,)), ((), ())),
                           precision=lax.Precision.HIGHEST,
                           preferred_element_type=jnp.float32)


def _tc_matmul(x, w1):
    def body(x_ref, w_ref, h_ref):
        h_ref[...] = _dot(x_ref[...], w_ref[...])

    return pl.pallas_call(
        body, out_shape=jax.ShapeDtypeStruct((N, D_H), jnp.float32))(x, w1)


def _tc_scale(h1, degacc):
    def body(h_ref, d_ref, hp_ref, dinv_ref):
        deg = 1.0 + d_ref[0, :N, 0:1] + d_ref[1, :N, 0:1]
        dinv = lax.rsqrt(deg)
        hp_ref[...] = h_ref[...] * dinv
        dinv_ref[...] = dinv

    return pl.pallas_call(
        body,
        out_shape=(jax.ShapeDtypeStruct((N, D_H), jnp.float32),
                   jax.ShapeDtypeStruct((N, 1), jnp.float32)),
    )(h1, degacc)


def _tc_mid(acc, hp, dinv, b, g, be, w_next, width_next):
    def body(a_ref, hp_ref, dinv_ref, b_ref, g_ref, be_ref, w_ref, out_ref):
        dinv = dinv_ref[...]
        z = dinv * (a_ref[0, :N, :] + a_ref[1, :N, :] + hp_ref[...]) + b_ref[...]
        mean = jnp.mean(z, axis=0, keepdims=True)
        zc = z - mean
        var = jnp.mean(zc * zc, axis=0, keepdims=True)
        r = g_ref[...] * zc * lax.rsqrt(var + 1e-5) + be_ref[...]
        r = jnp.maximum(r, 0.0)
        out_ref[...] = _dot(r, w_ref[...]) * dinv

    return pl.pallas_call(
        body,
        out_shape=jax.ShapeDtypeStruct((N, width_next), jnp.float32),
    )(acc, hp, dinv, b.reshape(1, -1), g.reshape(1, -1), be.reshape(1, -1), w_next)


def _tc_last(acc, hp, dinv, b3):
    def body(a_ref, hp_ref, dinv_ref, b_ref, out_ref):
        z = dinv_ref[...] * (a_ref[0, :N, :] + a_ref[1, :N, :] + hp_ref[...])
        out_ref[...] = z[:, :D_OUT] + b_ref[...]

    return pl.pallas_call(
        body,
        out_shape=jax.ShapeDtypeStruct((N, D_OUT), jnp.float32),
    )(acc, hp, dinv, b3.reshape(1, -1))


def kernel(x, edge_index, W1, b1, g1, be1, W2, b2, g2, be2, W3, b3):
    src = edge_index[0].astype(jnp.int32)
    dst = edge_index[1].astype(jnp.int32)
    pad = EPAD - E
    pad_src = jnp.arange(pad, dtype=jnp.int32) % N
    pad_dst = N + jnp.arange(pad, dtype=jnp.int32) % (NPAD - N)
    src_r = jnp.concatenate([src, pad_src])
    dst_r = jnp.concatenate([dst, pad_dst])
    dst4 = dst_r.reshape(NC, NS, CHUNKS_PER_SUB, CHUNK)
    src_r = src_r.reshape(NC, NS, NPHASE, HALF, CHUNK)
    dst_r = dst_r.reshape(NC, NS, NPHASE, HALF, CHUNK)

    zrows = jnp.zeros((ROWS_PER_SUB, D_H), jnp.float32)
    zdeg = jnp.zeros((ROWS_PER_SUB, DEG_W), jnp.float32)
    ones = jnp.ones((CHUNK, DEG_W), jnp.float32)

    degacc = _sc_degree()(dst4, zdeg, ones)
    h1 = _tc_matmul(x, W1)
    h1p, dinv = _tc_scale(h1, degacc)
    acc1 = _sc_scatter_rows(D_H)(h1p, src_r, dst_r, zrows)
    h2p = _tc_mid(acc1, h1p, dinv, b1, g1, be1, W2, D_H)
    acc2 = _sc_scatter_rows(D_H)(h2p, src_r, dst_r, zrows)
    w3p = jnp.concatenate([W3, jnp.zeros((D_H, D_OUT_PAD - D_OUT), jnp.float32)], axis=1)
    h3p = _tc_mid(acc2, h2p, dinv, b2, g2, be2, w3p, D_OUT_PAD)
    acc3 = _sc_scatter_rows(D_OUT_PAD)(h3p, src_r, dst_r, zrows)
    return _tc_last(acc3, h3p, dinv, b3)


# speedup vs baseline: 1.0440x; 1.0440x over previous
import functools

import jax
import jax.numpy as jnp
from jax import lax
from jax.experimental import pallas as pl
from jax.experimental.pallas import tpu as pltpu
from jax.experimental.pallas import tpu_sc as plsc

N = 10000
E = 320000
D_IN = 128
D_H = 128
D_OUT = 40
D_OUT_PAD = 128

NC = 2
NS = 16
CHUNK = 128
CHUNKS_PER_SUB = 80
NPHASE = 2
HALF = CHUNKS_PER_SUB // NPHASE
EPAD = NC * NS * CHUNKS_PER_SUB * CHUNK
NPAD = 10240
ROWS_PER_SUB = NPAD // NS
DEG_W = 16

_VMESH = plsc.VectorSubcoreMesh(core_axis_name="c", subcore_axis_name="s")


def _row_blocks():
    off = 0
    blocks = []
    while off < ROWS_PER_SUB:
        r = min(CHUNK, ROWS_PER_SUB - off)
        blocks.append((off, r))
        off += r
    return blocks


def _fill_rows(ref, value):
    rows, cols = ref.shape
    v = jnp.full((16,), value, ref.dtype)

    @pl.loop(0, rows)
    def _(i):
        for j in range(cols // 16):
            ref[i, pl.ds(j * 16, 16)] = v


@functools.lru_cache(maxsize=None)
def _sc_scatter_rows(width):
    @functools.partial(
        pl.kernel,
        out_type=jax.ShapeDtypeStruct((NC, NPAD, width), jnp.float32),
        mesh=_VMESH,
        scratch_types=[
            pltpu.VMEM((HALF, CHUNK), jnp.int32),
            pltpu.VMEM((HALF, CHUNK), jnp.int32),
            pltpu.VMEM((CHUNK, width), jnp.float32),
            pltpu.VMEM((CHUNK, width), jnp.float32),
            pltpu.VMEM_SHARED((NPAD, width), jnp.float32),
            pltpu.SemaphoreType.DMA,
            pltpu.SemaphoreType.DMA,
        ],
    )
    def k(h_hbm, src_hbm, dst_hbm, out_hbm, src_v, dst_v, buf0, buf1,
          acc, sem0, sem1):
        c = lax.axis_index("c")
        s = lax.axis_index("s")
        base = s * ROWS_PER_SUB

        def gather(j, buf, sem):
            pltpu.async_copy(h_hbm.at[src_v.at[j]], buf, sem)

        def gather_wait(buf, sem):
            pltpu.make_async_copy(h_hbm.at[src_v.at[0]], buf, sem).wait()

        _fill_rows(buf0, 0.0)
        for off, r in _row_blocks():
            pltpu.sync_copy(buf0.at[pl.ds(0, r)], acc.at[pl.ds(base + off, r)])
        plsc.subcore_barrier()

        for p in range(NPHASE):
            pltpu.sync_copy(src_hbm.at[c, s, p], src_v)
            pltpu.sync_copy(dst_hbm.at[c, s, p], dst_v)
            gather(0, buf0, sem0)
            gather(1, buf1, sem1)

            @pl.loop(0, HALF, step=2)
            def _(j):
                gather_wait(buf0, sem0)
                pltpu.sync_copy(buf0, acc.at[dst_v.at[j]], add=True)

                @pl.when(j + 2 < HALF)
                def _():
                    gather(j + 2, buf0, sem0)

                gather_wait(buf1, sem1)
                pltpu.sync_copy(buf1, acc.at[dst_v.at[j + 1]], add=True)

                @pl.when(j + 3 < HALF)
                def _():
                    gather(j + 3, buf1, sem1)

        plsc.subcore_barrier()
        pltpu.sync_copy(acc.at[pl.ds(base, ROWS_PER_SUB)],
                        out_hbm.at[c, pl.ds(base, ROWS_PER_SUB)])

    return k


@functools.lru_cache(maxsize=None)
def _sc_degree():
    @functools.partial(
        pl.kernel,
        out_type=jax.ShapeDtypeStruct((NC, NPAD, DEG_W), jnp.float32),
        mesh=_VMESH,
        scratch_types=[
            pltpu.VMEM((CHUNKS_PER_SUB, CHUNK), jnp.int32),
            pltpu.VMEM((CHUNK, DEG_W), jnp.float32),
            pltpu.VMEM((CHUNK, DEG_W), jnp.float32),
            pltpu.VMEM_SHARED((NPAD, DEG_W), jnp.float32),
        ],
    )
    def k(dst_hbm, out_hbm, dst_v, ones_v, buf_v, acc):
        c = lax.axis_index("c")
        s = lax.axis_index("s")
        base = s * ROWS_PER_SUB
        _fill_rows(buf_v, 0.0)
        for off, r in _row_blocks():
            pltpu.sync_copy(buf_v.at[pl.ds(0, r)], acc.at[pl.ds(base + off, r)])
        _fill_rows(ones_v, 1.0)
        pltpu.sync_copy(dst_hbm.at[c, s], dst_v)
        plsc.subcore_barrier()

        @pl.loop(0, CHUNKS_PER_SUB)
        def _(j):
            pltpu.sync_copy(ones_v, acc.at[dst_v.at[j]], add=True)

        plsc.subcore_barrier()
        pltpu.sync_copy(acc.at[pl.ds(base, ROWS_PER_SUB)],
                        out_hbm.at[c, pl.ds(base, ROWS_PER_SUB)])

    return k


def _dot(a, b):
    return lax.dot_general(a, b, (((1,), (0,)), ((), ())),
                           precision=lax.Precision.HIGHEST,
                           preferred_element_type=jnp.float32)


def _tc_matmul(x, w1):
    def body(x_ref, w_ref, h_ref):
        h_ref[...] = _dot(x_ref[...], w_ref[...])

    return pl.pallas_call(
        body, out_shape=jax.ShapeDtypeStruct((N, D_H), jnp.float32))(x, w1)


def _tc_scale(h1, degacc):
    def body(h_ref, d_ref, hp_ref, dinv_ref):
        deg = 1.0 + d_ref[0, :N, 0:1] + d_ref[1, :N, 0:1]
        dinv = lax.rsqrt(deg)
        hp_ref[...] = h_ref[...] * dinv
        dinv_ref[...] = dinv

    return pl.pallas_call(
        body,
        out_shape=(jax.ShapeDtypeStruct((N, D_H), jnp.float32),
                   jax.ShapeDtypeStruct((N, 1), jnp.float32)),
    )(h1, degacc)


def _tc_mid(acc, hp, dinv, b, g, be, w_next, width_next):
    def body(a_ref, hp_ref, dinv_ref, b_ref, g_ref, be_ref, w_ref, out_ref):
        dinv = dinv_ref[...]
        z = dinv * (a_ref[0, :N, :] + a_ref[1, :N, :] + hp_ref[...]) + b_ref[...]
        mean = jnp.mean(z, axis=0, keepdims=True)
        zc = z - mean
        var = jnp.mean(zc * zc, axis=0, keepdims=True)
        r = g_ref[...] * zc * lax.rsqrt(var + 1e-5) + be_ref[...]
        r = jnp.maximum(r, 0.0)
        out_ref[...] = _dot(r, w_ref[...]) * dinv

    return pl.pallas_call(
        body,
        out_shape=jax.ShapeDtypeStruct((N, width_next), jnp.float32),
    )(acc, hp, dinv, b.reshape(1, -1), g.reshape(1, -1), be.reshape(1, -1), w_next)


def _tc_last(acc, hp, dinv, b3):
    def body(a_ref, hp_ref, dinv_ref, b_ref, out_ref):
        z = dinv_ref[...] * (a_ref[0, :N, :] + a_ref[1, :N, :] + hp_ref[...])
        out_ref[...] = z[:, :D_OUT] + b_ref[...]

    return pl.pallas_call(
        body,
        out_shape=jax.ShapeDtypeStruct((N, D_OUT), jnp.float32),
    )(acc, hp, dinv, b3.reshape(1, -1))


def kernel(x, edge_index, W1, b1, g1, be1, W2, b2, g2, be2, W3, b3):
    src = edge_index[0].astype(jnp.int32)
    dst = edge_index[1].astype(jnp.int32)
    pad = EPAD - E
    pad_src = jnp.arange(pad, dtype=jnp.int32) % N
    pad_dst = N + jnp.arange(pad, dtype=jnp.int32) % (NPAD - N)
    src_r = jnp.concatenate([src, pad_src])
    dst_r = jnp.concatenate([dst, pad_dst])
    dst4 = dst_r.reshape(NC, NS, CHUNKS_PER_SUB, CHUNK)
    src_r = src_r.reshape(NC, NS, NPHASE, HALF, CHUNK)
    dst_r = dst_r.reshape(NC, NS, NPHASE, HALF, CHUNK)

    degacc = _sc_degree()(dst4)
    h1 = _tc_matmul(x, W1)
    h1p, dinv = _tc_scale(h1, degacc)
    acc1 = _sc_scatter_rows(D_H)(h1p, src_r, dst_r)
    h2p = _tc_mid(acc1, h1p, dinv, b1, g1, be1, W2, D_H)
    acc2 = _sc_scatter_rows(D_H)(h2p, src_r, dst_r)
    w3p = jnp.concatenate([W3, jnp.zeros((D_H, D_OUT_PAD - D_OUT), jnp.float32)], axis=1)
    h3p = _tc_mid(acc2, h2p, dinv, b2, g2, be2, w3p, D_OUT_PAD)
    acc3 = _sc_scatter_rows(D_OUT_PAD)(h3p, src_r, dst_r)
    return _tc_last(acc3, h3p, dinv, b3)
